# baseline (speedup 1.0000x reference)
import functools

import jax
import jax.numpy as jnp
from jax import lax
from jax.experimental import pallas as pl
from jax.experimental.pallas import tpu as pltpu
from jax.experimental.pallas import tpu_sc as plsc

NC = 2
NS = 16
NW = NC * NS
LANES = 16
CHUNK = 64
NBUF = 3
PREF = 1
PASSES = 2


def _sc_scatter(x, src, tgt, val, n, d, k_chunks):
    mesh = plsc.VectorSubcoreMesh(core_axis_name="c", subcore_axis_name="s")
    n_pad = -(-n // (NS * 8)) * (NS * 8)
    rows_per_tile = n_pad // NS

    kp0 = -(-(k_chunks // PASSES) // NBUF) * NBUF
    pass_sizes = [kp0, k_chunks - kp0] if PASSES == 2 else [k_chunks]
    k_stage = max(pass_sizes)

    @functools.partial(
        pl.kernel,
        out_type=jax.ShapeDtypeStruct((NC, n_pad, d), jnp.float32),
        mesh=mesh,
        compiler_params=pltpu.CompilerParams(use_tc_tiling_on_sc=False),
        scratch_types=[
            pltpu.VMEM((k_stage, CHUNK), jnp.int32),
            pltpu.VMEM((k_stage, CHUNK), jnp.int32),
            pltpu.VMEM((k_stage, CHUNK), jnp.float32),
            pltpu.VMEM((NBUF, CHUNK, d), jnp.float32),
            pltpu.VMEM_SHARED((n_pad, d), jnp.float32),
            pltpu.SemaphoreType.DMA((NBUF,)),
            pltpu.SemaphoreType.DMA((NBUF,)),
        ],
    )
    def body(x_hbm, src_hbm, tgt_hbm, val_hbm, out_hbm,
             src_v, tgt_v, val_v, rows_v, acc, gsem, ssem):
        cid = lax.axis_index("c")
        sid = lax.axis_index("s")
        wid = sid * NC + cid

        def stage(p_start, kp):
            pltpu.sync_copy(src_hbm.at[wid, pl.ds(p_start, kp)],
                            src_v.at[pl.ds(0, kp)])
            pltpu.sync_copy(tgt_hbm.at[wid, pl.ds(p_start, kp)],
                            tgt_v.at[pl.ds(0, kp)])
            pltpu.sync_copy(val_hbm.at[wid, pl.ds(p_start, kp)],
                            val_v.at[pl.ds(0, kp)])

        zeros = jnp.zeros((LANES,), jnp.float32)

        def zrow(i, _):
            for q in range(d // LANES):
                rows_v[0, i, pl.ds(q * LANES, LANES)] = zeros
            return 0

        lax.fori_loop(0, CHUNK, zrow, 0)

        base = sid * rows_per_tile
        full = rows_per_tile // CHUNK
        rem = rows_per_tile - full * CHUNK
        for t in range(full):
            pltpu.sync_copy(rows_v.at[0], acc.at[pl.ds(base + t * CHUNK, CHUNK)])
        if rem:
            pltpu.sync_copy(rows_v.at[0, pl.ds(0, rem)],
                            acc.at[pl.ds(base + full * CHUNK, rem)])
        plsc.subcore_barrier()

        def g_start(k, b):
            pltpu.async_copy(x_hbm.at[src_v.at[k]], rows_v.at[b], gsem.at[b])

        def g_wait(b):
            pltpu.make_async_copy(x_hbm.at[src_v.at[0]], rows_v.at[b],
                                  gsem.at[b]).wait()

        def s_start(k, b):
            pltpu.async_copy(rows_v.at[b], acc.at[tgt_v.at[k]], ssem.at[b],
                             add=True)

        def s_wait(b):
            pltpu.make_async_copy(rows_v.at[b], acc.at[tgt_v.at[0]],
                                  ssem.at[b]).wait()

        def scale(k, b):
            def edge_group(g, _):
                vals = val_v[k, pl.ds(g * LANES, LANES)]
                for t in range(LANES):
                    v = vals[t]
                    row = g * LANES + t
                    for q in range(d // LANES):
                        sl = pl.ds(q * LANES, LANES)
                        rows_v[b, row, sl] = rows_v[b, row, sl] * v
                return 0

            lax.fori_loop(0, CHUNK // LANES, edge_group, 0)

        p_start = 0
        for kp in pass_sizes:
            stage(p_start, kp)
            for j in range(PREF):
                g_start(j, j % NBUF)

            def outer(kk, _):
                for b0 in range(NBUF):
                    k = kk * NBUF + b0

                    @pl.when(k >= NBUF - PREF)
                    def _():
                        s_wait((b0 + PREF) % NBUF)

                    @pl.when(k + PREF < kp)
                    def _():
                        g_start(k + PREF, (b0 + PREF) % NBUF)

                    g_wait(b0)
                    scale(k, b0)
                    s_start(k, b0)
                return 0

            lax.fori_loop(0, kp // NBUF, outer, 0)
            for j in range(NBUF - PREF):
                s_wait((kp - (NBUF - PREF) + j) % NBUF)
            p_start += kp
        plsc.subcore_barrier()

        pltpu.sync_copy(acc.at[pl.ds(base, rows_per_tile)],
                        out_hbm.at[cid, pl.ds(base, rows_per_tile)])

    return body(x, src, tgt, val)


def _tc_add(a, b):
    n, d = a.shape

    def add_body(a_ref, b_ref, o_ref):
        o_ref[...] = a_ref[...] + b_ref[...]

    return pl.pallas_call(
        add_body,
        out_shape=jax.ShapeDtypeStruct((n, d), jnp.float32),
    )(a, b)


def kernel(x, edge_index, edge_values):
    n, d = x.shape
    e = edge_values.shape[0]
    per_tile = -(-e // NW)
    k_chunks = -(-per_tile // CHUNK)
    k_chunks = max(-(-k_chunks // NBUF) * NBUF, NBUF * PASSES)
    e_pad = NW * k_chunks * CHUNK

    tgt = edge_index[0]
    src = edge_index[1]
    pad = e_pad - e
    if pad:
        src = jnp.concatenate([src, jnp.zeros((pad,), jnp.int32)])
        tgt = jnp.concatenate([tgt, jnp.zeros((pad,), jnp.int32)])
        edge_values = jnp.concatenate([edge_values, jnp.zeros((pad,), jnp.float32)])
    src = src.reshape(NW, k_chunks, CHUNK)
    tgt = tgt.reshape(NW, k_chunks, CHUNK)
    val = edge_values.reshape(NW, k_chunks, CHUNK)

    partials = _sc_scatter(x, src, tgt, val, n, d, k_chunks)
    return _tc_add(partials[0, :n], partials[1, :n])


# speedup vs baseline: 1.8234x; 1.8234x over previous
import functools

import jax
import jax.numpy as jnp
from jax import lax
from jax.experimental import pallas as pl
from jax.experimental.pallas import tpu as pltpu
from jax.experimental.pallas import tpu_sc as plsc

NC = 2
NS = 16
LANES = 16
CHUNK = 112
NBUF = 4
PREF = 2


def _sc_scatter(xs, src, tgt, val, n, d, k_chunks):
    mesh = plsc.VectorSubcoreMesh(core_axis_name="c", subcore_axis_name="s")
    dh = d // NC
    n_pad = -(-n // (NS * 8)) * (NS * 8)
    rows_per_tile = n_pad // NS

    @functools.partial(
        pl.kernel,
        out_type=jax.ShapeDtypeStruct((NC, n_pad, dh), jnp.float32),
        mesh=mesh,
        compiler_params=pltpu.CompilerParams(use_tc_tiling_on_sc=False),
        scratch_types=[
            pltpu.VMEM((k_chunks, CHUNK), jnp.int32),
            pltpu.VMEM((k_chunks, CHUNK), jnp.int32),
            pltpu.VMEM((k_chunks, CHUNK), jnp.float32),
            pltpu.VMEM((NBUF, CHUNK, dh), jnp.float32),
            pltpu.VMEM_SHARED((n_pad, dh), jnp.float32),
            pltpu.SemaphoreType.DMA((NBUF,)),
            pltpu.SemaphoreType.DMA((NBUF,)),
        ],
    )
    def body(xs_hbm, src_hbm, tgt_hbm, val_hbm, out_hbm,
             src_v, tgt_v, val_v, rows_v, acc, gsem, ssem):
        cid = lax.axis_index("c")
        sid = lax.axis_index("s")

        pltpu.sync_copy(src_hbm.at[sid], src_v)
        pltpu.sync_copy(tgt_hbm.at[sid], tgt_v)
        pltpu.sync_copy(val_hbm.at[sid], val_v)

        off = cid * n

        def offs(k, _):
            for g in range(CHUNK // LANES):
                sl = pl.ds(g * LANES, LANES)
                src_v[k, sl] = src_v[k, sl] + off
            return 0

        lax.fori_loop(0, k_chunks, offs, 0)

        zeros = jnp.zeros((LANES,), jnp.float32)

        def zrow(i, _):
            for q in range(dh // LANES):
                rows_v[0, i, pl.ds(q * LANES, LANES)] = zeros
            return 0

        lax.fori_loop(0, CHUNK, zrow, 0)

        base = sid * rows_per_tile
        full = rows_per_tile // CHUNK
        rem = rows_per_tile - full * CHUNK
        for t in range(full):
            pltpu.sync_copy(rows_v.at[0], acc.at[pl.ds(base + t * CHUNK, CHUNK)])
        if rem:
            pltpu.sync_copy(rows_v.at[0, pl.ds(0, rem)],
                            acc.at[pl.ds(base + full * CHUNK, rem)])
        plsc.subcore_barrier()

        def g_start(k, b):
            pltpu.async_copy(xs_hbm.at[src_v.at[k]], rows_v.at[b], gsem.at[b])

        def g_wait(b):
            pltpu.make_async_copy(xs_hbm.at[src_v.at[0]], rows_v.at[b],
                                  gsem.at[b]).wait()

        def s_start(k, b):
            pltpu.async_copy(rows_v.at[b], acc.at[tgt_v.at[k]], ssem.at[b],
                             add=True)

        def s_wait(b):
            pltpu.make_async_copy(rows_v.at[b], acc.at[tgt_v.at[0]],
                                  ssem.at[b]).wait()

        def scale(k, b):
            def edge_group(g, _):
                vals = val_v[k, pl.ds(g * LANES, LANES)]
                for t in range(LANES):
                    v = vals[t]
                    row = g * LANES + t
                    for q in range(dh // LANES):
                        sl = pl.ds(q * LANES, LANES)
                        rows_v[b, row, sl] = rows_v[b, row, sl] * v
                return 0

            lax.fori_loop(0, CHUNK // LANES, edge_group, 0)

        for j in range(PREF):
            g_start(j, j % NBUF)

        def outer(kk, _):
            for b0 in range(NBUF):
                k = kk * NBUF + b0

                @pl.when(k >= NBUF - PREF)
                def _():
                    s_wait((b0 + PREF) % NBUF)

                @pl.when(k + PREF < k_chunks)
                def _():
                    g_start(k + PREF, (b0 + PREF) % NBUF)

                g_wait(b0)
                scale(k, b0)
                s_start(k, b0)
            return 0

        lax.fori_loop(0, k_chunks // NBUF, outer, 0)
        for j in range(NBUF - PREF):
            s_wait((k_chunks - (NBUF - PREF) + j) % NBUF)
        plsc.subcore_barrier()

        pltpu.sync_copy(acc.at[pl.ds(base, rows_per_tile)],
                        out_hbm.at[cid, pl.ds(base, rows_per_tile)])

    return body(xs, src, tgt, val)


def kernel(x, edge_index, edge_values):
    n, d = x.shape
    e = edge_values.shape[0]
    dh = d // NC
    per_tile = -(-e // NS)
    k_chunks = -(-per_tile // CHUNK)
    k_chunks = max(-(-k_chunks // NBUF) * NBUF, NBUF)
    e_pad = NS * k_chunks * CHUNK

    xs = jnp.transpose(x.reshape(n, NC, dh), (1, 0, 2)).reshape(NC * n, dh)

    tgt = edge_index[0]
    src = edge_index[1]
    pad = e_pad - e
    if pad:
        src = jnp.concatenate([src, jnp.zeros((pad,), jnp.int32)])
        tgt = jnp.concatenate([tgt, jnp.zeros((pad,), jnp.int32)])
        edge_values = jnp.concatenate([edge_values, jnp.zeros((pad,), jnp.float32)])
    src = src.reshape(NS, k_chunks, CHUNK)
    tgt = tgt.reshape(NS, k_chunks, CHUNK)
    val = edge_values.reshape(NS, k_chunks, CHUNK)

    halves = _sc_scatter(xs, src, tgt, val, n, d, k_chunks)
    return jnp.concatenate([halves[0, :n], halves[1, :n]], axis=1)
